# baseline (speedup 1.0000x reference)
import jax
import jax.numpy as jnp
from jax import lax
from jax.experimental import pallas as pl
from jax.experimental.pallas import tpu as pltpu
from jax.experimental.pallas import tpu_sc as plsc

N_NODES = 10000
N_EDGES = 320000
D = 128

NC = 2
NS = 16
NW = NC * NS
LANES = 16

N_PAD = 10240
E_PAD = 327680
EDGES_PER_W = E_PAD // NW
CHUNK = 32
NCHUNKS = EDGES_PER_W // CHUNK
SUP = 64
NSUP = NCHUNKS // SUP
SPAIRS = SUP // 2
ROWS_PER_S = N_PAD // NS

def _psr_body(nodes_ref, w_ref, b_ref, ps_ref, pr_ref):
    n = nodes_ref[...]
    w = w_ref[...]
    ps_ref[...] = (
        jnp.dot(n, w[0:D, :], preferred_element_type=jnp.float32) + b_ref[...]
    )
    pr_ref[...] = jnp.dot(n, w[D : 2 * D, :], preferred_element_type=jnp.float32)


def _e_body(edges_ref, w_ref, e_ref):
    e = jnp.dot(edges_ref[...], w_ref[2 * D :, :], preferred_element_type=jnp.float32)
    lo = jnp.concatenate([e[:, 32 * g : 32 * g + 16] for g in range(4)], axis=1)
    hi = jnp.concatenate([e[:, 32 * g + 16 : 32 * g + 32] for g in range(4)], axis=1)
    lo16 = jax.lax.bitcast_convert_type(lo.astype(jnp.bfloat16), jnp.uint16)
    hi16 = jax.lax.bitcast_convert_type(hi.astype(jnp.bfloat16), jnp.uint16)
    packed = jax.lax.shift_left(hi16.astype(jnp.uint32), jnp.uint32(16)) | lo16.astype(jnp.uint32)
    e_ref[...] = jax.lax.bitcast_convert_type(packed, jnp.int32)


def _final_body(nodes_ref, agg_ref, w1_ref, b1_ref, w2_ref, b2_ref, out_ref):
    n = nodes_ref[...]
    agg = agg_ref[0] + agg_ref[1]
    h = jnp.dot(n, w1_ref[0:D, :], preferred_element_type=jnp.float32)
    h = h + jnp.dot(agg, w1_ref[D:, :], preferred_element_type=jnp.float32)
    h = jnp.maximum(h + b1_ref[...], 0.0)
    h = jnp.dot(h, w2_ref[...], preferred_element_type=jnp.float32) + b2_ref[...]
    out_ref[...] = n + h


def _sc_body(
    ps_hbm, pr_hbm, e_hbm, snd_hbm, rcv_hbm, z_hbm, out_hbm,
    agg_sh, sidx, ridx, ps_v, pr_v, e_v, m_v, sems, sem_sc,
):
    c = lax.axis_index("c")
    s = lax.axis_index("s")
    w = s * NC + c

    pltpu.sync_copy(z_hbm, agg_sh.at[pl.ds(s * ROWS_PER_S, ROWS_PER_S)])
    plsc.subcore_barrier()

    def _issue(sj, t, b):
        jg = sj * SUP + t
        pltpu.async_copy(ps_hbm.at[sidx.at[t]], ps_v.at[b], sems.at[b])
        pltpu.async_copy(pr_hbm.at[ridx.at[t]], pr_v.at[b], sems.at[b])
        pltpu.async_copy(e_hbm.at[w, jg], e_v.at[b], sems.at[b])

    def _drain(sj, t, b):
        jg = sj * SUP + t
        pltpu.make_async_copy(ps_hbm.at[sidx.at[t]], ps_v.at[b], sems.at[b]).wait()
        pltpu.make_async_copy(pr_hbm.at[ridx.at[t]], pr_v.at[b], sems.at[b]).wait()
        pltpu.make_async_copy(e_hbm.at[w, jg], e_v.at[b], sems.at[b]).wait()

    def _scatter(t, b):
        pltpu.async_copy(m_v.at[b], agg_sh.at[ridx.at[t]], sem_sc.at[b], add=True)

    def _wait_scatter(t, b):
        pltpu.make_async_copy(m_v.at[b], agg_sh.at[ridx.at[t]], sem_sc.at[b]).wait()

    def _compute(b):
        def _rows(i2, _):
            base = pl.multiple_of(i2 * 2, 2)
            for di in range(2):
                i = base + di
                for g in range(D // (2 * LANES)):
                    sl_lo = pl.ds(g * 2 * LANES, LANES)
                    sl_hi = pl.ds(g * 2 * LANES + LANES, LANES)
                    ew = e_v[b, i, pl.ds(g * LANES, LANES)]
                    e_lo = jax.lax.bitcast_convert_type(
                        jax.lax.shift_left(ew, 16), jnp.float32
                    )
                    e_hi = jax.lax.bitcast_convert_type(
                        jnp.bitwise_and(ew, jnp.int32(-65536)), jnp.float32
                    )
                    m_v[b, i, sl_lo] = jnp.maximum(
                        ps_v[b, i, sl_lo] + pr_v[b, i, sl_lo] + e_lo, 0.0
                    )
                    m_v[b, i, sl_hi] = jnp.maximum(
                        ps_v[b, i, sl_hi] + pr_v[b, i, sl_hi] + e_hi, 0.0
                    )
            return 0

        lax.fori_loop(0, CHUNK // 2, _rows, 0)

    for sj in range(NSUP):
        pltpu.sync_copy(snd_hbm.at[w, sj], sidx)
        pltpu.sync_copy(rcv_hbm.at[w, sj], ridx)
        _issue(sj, 0, 0)
        _issue(sj, 1, 1)

        def _kbody(k, _, sj=sj):
            for b in (0, 1):
                t = 2 * k + b
                _drain(sj, t, b)

                @pl.when(k > 0)
                def _():
                    _wait_scatter(t - 2, b)

                _compute(b)
                _scatter(t, b)

                @pl.when(k < SPAIRS - 1)
                def _():
                    _issue(sj, t + 2, b)
            return 0

        lax.fori_loop(0, SPAIRS, _kbody, 0)
        _wait_scatter(SUP - 2, 0)
        _wait_scatter(SUP - 1, 1)

    plsc.subcore_barrier()

    pltpu.sync_copy(
        agg_sh.at[pl.ds(s * ROWS_PER_S, ROWS_PER_S)],
        out_hbm.at[c, pl.ds(s * ROWS_PER_S, ROWS_PER_S)],
    )


def _segment_messages(ps, pr, e, senders, receivers):
    mesh = plsc.VectorSubcoreMesh(
        core_axis_name="c", subcore_axis_name="s", num_cores=NC, num_subcores=NS
    )
    return pl.kernel(
        _sc_body,
        out_type=jax.ShapeDtypeStruct((NC, N_PAD, D), jnp.float32),
        mesh=mesh,
        scratch_types=[
            pltpu.VMEM_SHARED((N_PAD, D), jnp.float32),
            pltpu.VMEM((SUP, CHUNK), jnp.int32),
            pltpu.VMEM((SUP, CHUNK), jnp.int32),
            pltpu.VMEM((2, CHUNK, D), jnp.float32),
            pltpu.VMEM((2, CHUNK, D), jnp.float32),
            pltpu.VMEM((2, CHUNK, D // 2), jnp.int32),
            pltpu.VMEM((2, CHUNK, D), jnp.float32),
            pltpu.SemaphoreType.DMA((2,)),
            pltpu.SemaphoreType.DMA((2,)),
        ],
    )(
        ps,
        pr,
        e.reshape(NW, NCHUNKS, CHUNK, D // 2),
        senders.reshape(NW, NSUP, SUP, CHUNK),
        receivers.reshape(NW, NSUP, SUP, CHUNK),
        jnp.zeros((ROWS_PER_S, D), jnp.float32),
    )


def _pad_edges(edges, senders, receivers):
    npad = E_PAD - N_EDGES
    edges_p = jnp.concatenate([edges, jnp.zeros((npad, 16), edges.dtype)])
    senders_p = jnp.concatenate([senders, jnp.zeros((npad,), senders.dtype)])
    receivers_p = jnp.concatenate(
        [receivers, jnp.full((npad,), N_PAD - 1, receivers.dtype)]
    )
    return edges_p, senders_p, receivers_p


def kernel(nodes, edges, senders, receivers, W_msg, b_msg, W_n1, b_n1, W_n2, b_n2):
    edges, senders, receivers = _pad_edges(edges, senders, receivers)
    b_msg2 = b_msg.reshape(1, D)
    b1 = b_n1.reshape(1, D)
    b2 = b_n2.reshape(1, D)

    ps, pr = pl.pallas_call(
        _psr_body,
        out_shape=(
            jax.ShapeDtypeStruct((N_NODES, D), jnp.float32),
            jax.ShapeDtypeStruct((N_NODES, D), jnp.float32),
        ),
    )(nodes, W_msg, b_msg2)

    eblk = 8192
    e = pl.pallas_call(
        _e_body,
        grid=(E_PAD // eblk,),
        in_specs=[
            pl.BlockSpec((eblk, 16), lambda i: (i, 0)),
            pl.BlockSpec((2 * D + 16, D), lambda i: (0, 0)),
        ],
        out_specs=pl.BlockSpec((eblk, D // 2), lambda i: (i, 0)),
        out_shape=jax.ShapeDtypeStruct((E_PAD, D // 2), jnp.int32),
    )(edges, W_msg)

    agg2 = _segment_messages(ps, pr, e, senders, receivers)[:, :N_NODES, :]

    nblk = 1000
    out = pl.pallas_call(
        _final_body,
        grid=(N_NODES // nblk,),
        in_specs=[
            pl.BlockSpec((nblk, D), lambda i: (i, 0)),
            pl.BlockSpec((NC, nblk, D), lambda i: (0, i, 0)),
            pl.BlockSpec((2 * D, D), lambda i: (0, 0)),
            pl.BlockSpec((1, D), lambda i: (0, 0)),
            pl.BlockSpec((D, D), lambda i: (0, 0)),
            pl.BlockSpec((1, D), lambda i: (0, 0)),
        ],
        out_specs=pl.BlockSpec((nblk, D), lambda i: (i, 0)),
        out_shape=jax.ShapeDtypeStruct((N_NODES, D), jnp.float32),
    )(nodes, agg2, W_n1, b1, W_n2, b2)
    return out


# speedup vs baseline: 1.0426x; 1.0426x over previous
import jax
import jax.numpy as jnp
import numpy as np
from jax import lax
from jax.experimental import pallas as pl
from jax.experimental.pallas import tpu as pltpu
from jax.experimental.pallas import tpu_sc as plsc

N_NODES = 10000
N_EDGES = 320000
D = 128

NC = 2
NS = 16
NW = NC * NS
LANES = 16

N_PAD = 10240
E_PAD = 327680
EDGES_PER_W = E_PAD // NW
CHUNK = 32
NCHUNKS = EDGES_PER_W // CHUNK
SUP = 64
NSUP = NCHUNKS // SUP
SPAIRS = SUP // 2
ROWS_PER_S = N_PAD // NS

def _psr_body(nodes_ref, w_ref, b_ref, ps_ref, pr_ref):
    n = nodes_ref[...]
    w = w_ref[...]
    ps_ref[...] = (
        jnp.dot(n, w[0:D, :], preferred_element_type=jnp.float32) + b_ref[...]
    )
    pr_ref[...] = jnp.dot(n, w[D : 2 * D, :], preferred_element_type=jnp.float32)


def _e_body(edges_ref, w_ref, e_ref):
    e = jnp.dot(edges_ref[...], w_ref[...], preferred_element_type=jnp.float32)
    lo = e[:, : D // 2]
    hi = e[:, D // 2 :]
    lo16 = jax.lax.bitcast_convert_type(lo.astype(jnp.bfloat16), jnp.uint16)
    hi16 = jax.lax.bitcast_convert_type(hi.astype(jnp.bfloat16), jnp.uint16)
    packed = (
        jax.lax.shift_left(hi16.astype(jnp.uint32), jnp.uint32(16))
        | lo16.astype(jnp.uint32)
    )
    e_ref[...] = jax.lax.bitcast_convert_type(packed, jnp.int32)


def _final_body(nodes_ref, agg_ref, w1_ref, b1_ref, w2_ref, b2_ref, out_ref):
    n = nodes_ref[...]
    agg = agg_ref[0] + agg_ref[1]
    h = jnp.dot(n, w1_ref[0:D, :], preferred_element_type=jnp.float32)
    h = h + jnp.dot(agg, w1_ref[D:, :], preferred_element_type=jnp.float32)
    h = jnp.maximum(h + b1_ref[...], 0.0)
    h = jnp.dot(h, w2_ref[...], preferred_element_type=jnp.float32) + b2_ref[...]
    out_ref[...] = n + h


def _sc_body(
    ps_hbm, pr_hbm, e_hbm, snd_hbm, rcv_hbm, z_hbm, out_hbm,
    agg_sh, sidx, ridx, ps_v, pr_v, e_v, m_v, sems, sem_sc,
):
    c = lax.axis_index("c")
    s = lax.axis_index("s")
    w = s * NC + c

    pltpu.sync_copy(z_hbm, agg_sh.at[pl.ds(s * ROWS_PER_S, ROWS_PER_S)])
    plsc.subcore_barrier()

    def _issue(sj, t, b):
        jg = sj * SUP + t
        pltpu.async_copy(ps_hbm.at[sidx.at[t]], ps_v.at[b], sems.at[b])
        pltpu.async_copy(pr_hbm.at[ridx.at[t]], pr_v.at[b], sems.at[b])
        pltpu.async_copy(e_hbm.at[w, jg], e_v.at[b], sems.at[b])

    def _drain(sj, t, b):
        jg = sj * SUP + t
        pltpu.make_async_copy(ps_hbm.at[sidx.at[t]], ps_v.at[b], sems.at[b]).wait()
        pltpu.make_async_copy(pr_hbm.at[ridx.at[t]], pr_v.at[b], sems.at[b]).wait()
        pltpu.make_async_copy(e_hbm.at[w, jg], e_v.at[b], sems.at[b]).wait()

    def _scatter(t, b):
        pltpu.async_copy(m_v.at[b], agg_sh.at[ridx.at[t]], sem_sc.at[b], add=True)

    def _wait_scatter(t, b):
        pltpu.make_async_copy(m_v.at[b], agg_sh.at[ridx.at[t]], sem_sc.at[b]).wait()

    def _compute(b):
        def _rows(i2, _):
            base = pl.multiple_of(i2 * 2, 2)
            for di in range(2):
                i = base + di
                for g in range(D // (2 * LANES)):
                    sl_lo = pl.ds(g * 2 * LANES, LANES)
                    sl_hi = pl.ds(g * 2 * LANES + LANES, LANES)
                    ew = e_v[b, i, pl.ds(g * LANES, LANES)]
                    e_lo = jax.lax.bitcast_convert_type(
                        jax.lax.shift_left(ew, 16), jnp.float32
                    )
                    e_hi = jax.lax.bitcast_convert_type(
                        jnp.bitwise_and(ew, jnp.int32(-65536)), jnp.float32
                    )
                    m_v[b, i, sl_lo] = jnp.maximum(
                        ps_v[b, i, sl_lo] + pr_v[b, i, sl_lo] + e_lo, 0.0
                    )
                    m_v[b, i, sl_hi] = jnp.maximum(
                        ps_v[b, i, sl_hi] + pr_v[b, i, sl_hi] + e_hi, 0.0
                    )
            return 0

        lax.fori_loop(0, CHUNK // 2, _rows, 0)

    for sj in range(NSUP):
        pltpu.sync_copy(snd_hbm.at[w, sj], sidx)
        pltpu.sync_copy(rcv_hbm.at[w, sj], ridx)
        _issue(sj, 0, 0)
        _issue(sj, 1, 1)

        def _kbody(k, _, sj=sj):
            for b in (0, 1):
                t = 2 * k + b
                _drain(sj, t, b)

                @pl.when(k > 0)
                def _():
                    _wait_scatter(t - 2, b)

                _compute(b)
                _scatter(t, b)

                @pl.when(k < SPAIRS - 1)
                def _():
                    _issue(sj, t + 2, b)
            return 0

        lax.fori_loop(0, SPAIRS, _kbody, 0)
        _wait_scatter(SUP - 2, 0)
        _wait_scatter(SUP - 1, 1)

    plsc.subcore_barrier()

    pltpu.sync_copy(
        agg_sh.at[pl.ds(s * ROWS_PER_S, ROWS_PER_S)],
        out_hbm.at[c, pl.ds(s * ROWS_PER_S, ROWS_PER_S)],
    )


def _segment_messages(ps, pr, e, senders, receivers):
    mesh = plsc.VectorSubcoreMesh(
        core_axis_name="c", subcore_axis_name="s", num_cores=NC, num_subcores=NS
    )
    return pl.kernel(
        _sc_body,
        out_type=jax.ShapeDtypeStruct((NC, N_PAD, D), jnp.float32),
        mesh=mesh,
        scratch_types=[
            pltpu.VMEM_SHARED((N_PAD, D), jnp.float32),
            pltpu.VMEM((SUP, CHUNK), jnp.int32),
            pltpu.VMEM((SUP, CHUNK), jnp.int32),
            pltpu.VMEM((2, CHUNK, D), jnp.float32),
            pltpu.VMEM((2, CHUNK, D), jnp.float32),
            pltpu.VMEM((2, CHUNK, D // 2), jnp.int32),
            pltpu.VMEM((2, CHUNK, D), jnp.float32),
            pltpu.SemaphoreType.DMA((2,)),
            pltpu.SemaphoreType.DMA((2,)),
        ],
    )(
        ps,
        pr,
        e.reshape(NW, NCHUNKS, CHUNK, D // 2),
        senders.reshape(NW, NSUP, SUP, CHUNK),
        receivers.reshape(NW, NSUP, SUP, CHUNK),
        jnp.zeros((ROWS_PER_S, D), jnp.float32),
    )


def _pad_edges(edges, senders, receivers):
    npad = E_PAD - N_EDGES
    edges_p = jnp.concatenate([edges, jnp.zeros((npad, 16), edges.dtype)])
    senders_p = jnp.concatenate([senders, jnp.zeros((npad,), senders.dtype)])
    receivers_p = jnp.concatenate(
        [receivers, jnp.full((npad,), N_PAD - 1, receivers.dtype)]
    )
    return edges_p, senders_p, receivers_p


def kernel(nodes, edges, senders, receivers, W_msg, b_msg, W_n1, b_n1, W_n2, b_n2):
    edges, senders, receivers = _pad_edges(edges, senders, receivers)
    b_msg2 = b_msg.reshape(1, D)
    lo_cols = np.concatenate([np.arange(16) + 32 * g for g in range(4)])
    w_e = W_msg[2 * D :, :]
    w_e_re = jnp.concatenate(
        [jnp.take(w_e, jnp.asarray(lo_cols), axis=1),
         jnp.take(w_e, jnp.asarray(lo_cols + 16), axis=1)],
        axis=1,
    )
    b1 = b_n1.reshape(1, D)
    b2 = b_n2.reshape(1, D)

    ps, pr = pl.pallas_call(
        _psr_body,
        out_shape=(
            jax.ShapeDtypeStruct((N_NODES, D), jnp.float32),
            jax.ShapeDtypeStruct((N_NODES, D), jnp.float32),
        ),
    )(nodes, W_msg, b_msg2)

    eblk = 8192
    e = pl.pallas_call(
        _e_body,
        grid=(E_PAD // eblk,),
        in_specs=[
            pl.BlockSpec((eblk, 16), lambda i: (i, 0)),
            pl.BlockSpec((16, D), lambda i: (0, 0)),
        ],
        out_specs=pl.BlockSpec((eblk, D // 2), lambda i: (i, 0)),
        out_shape=jax.ShapeDtypeStruct((E_PAD, D // 2), jnp.int32),
    )(edges, w_e_re)

    agg2 = _segment_messages(ps, pr, e, senders, receivers)[:, :N_NODES, :]

    nblk = 1000
    out = pl.pallas_call(
        _final_body,
        grid=(N_NODES // nblk,),
        in_specs=[
            pl.BlockSpec((nblk, D), lambda i: (i, 0)),
            pl.BlockSpec((NC, nblk, D), lambda i: (0, i, 0)),
            pl.BlockSpec((2 * D, D), lambda i: (0, 0)),
            pl.BlockSpec((1, D), lambda i: (0, 0)),
            pl.BlockSpec((D, D), lambda i: (0, 0)),
            pl.BlockSpec((1, D), lambda i: (0, 0)),
        ],
        out_specs=pl.BlockSpec((nblk, D), lambda i: (i, 0)),
        out_shape=jax.ShapeDtypeStruct((N_NODES, D), jnp.float32),
    )(nodes, agg2, W_n1, b1, W_n2, b2)
    return out


# speedup vs baseline: 1.1041x; 1.0589x over previous
import jax
import jax.numpy as jnp
import numpy as np
from jax import lax
from jax.experimental import pallas as pl
from jax.experimental.pallas import tpu as pltpu
from jax.experimental.pallas import tpu_sc as plsc

N_NODES = 10000
N_EDGES = 320000
D = 128

NC = 2
NS = 16
NW = NC * NS
LANES = 16

N_PAD = 10240
E_PAD = 327680
EDGES_PER_W = E_PAD // NW
CHUNK = 32
NCHUNKS = EDGES_PER_W // CHUNK
SUP = 64
NSUP = NCHUNKS // SUP
SPAIRS = SUP // 2
ROWS_PER_S = N_PAD // NS

def _pack_cols(x):
    lo16 = jax.lax.bitcast_convert_type(
        x[:, : D // 2].astype(jnp.bfloat16), jnp.uint16
    )
    hi16 = jax.lax.bitcast_convert_type(
        x[:, D // 2 :].astype(jnp.bfloat16), jnp.uint16
    )
    packed = (
        jax.lax.shift_left(hi16.astype(jnp.uint32), jnp.uint32(16))
        | lo16.astype(jnp.uint32)
    )
    return jax.lax.bitcast_convert_type(packed, jnp.int32)


def _psr_body(nodes_ref, w_ref, b_ref, ps_ref, pr_ref):
    n = nodes_ref[...]
    w = w_ref[...]
    ps = jnp.dot(n, w[0:D, :], preferred_element_type=jnp.float32) + b_ref[...]
    ps_ref[...] = _pack_cols(ps)
    pr = jnp.dot(n, w[D : 2 * D, :], preferred_element_type=jnp.float32)
    pr_ref[...] = _pack_cols(pr)


def _e_body(edges_ref, w_ref, e_ref):
    e = jnp.dot(edges_ref[...], w_ref[...], preferred_element_type=jnp.float32)
    e_ref[...] = _pack_cols(e)


def _final_body(nodes_ref, agg_ref, w1_ref, b1_ref, w2_ref, b2_ref, out_ref):
    n = nodes_ref[...]
    agg = agg_ref[0] + agg_ref[1]
    h = jnp.dot(n, w1_ref[0:D, :], preferred_element_type=jnp.float32)
    h = h + jnp.dot(agg, w1_ref[D:, :], preferred_element_type=jnp.float32)
    h = jnp.maximum(h + b1_ref[...], 0.0)
    h = jnp.dot(h, w2_ref[...], preferred_element_type=jnp.float32) + b2_ref[...]
    out_ref[...] = n + h


def _xlo(w):
    return jax.lax.bitcast_convert_type(jax.lax.shift_left(w, 16), jnp.float32)


def _xhi(w):
    return jax.lax.bitcast_convert_type(
        jnp.bitwise_and(w, jnp.int32(-65536)), jnp.float32
    )


def _sc_body(
    ps_hbm, pr_hbm, e_hbm, snd_hbm, rcv_hbm, z_hbm, out_hbm,
    agg_sh, sidx, ridx, ps_v, pr_v, e_v, m_v, sems, sem_sc,
):
    c = lax.axis_index("c")
    s = lax.axis_index("s")
    w = s * NC + c

    pltpu.sync_copy(z_hbm, agg_sh.at[pl.ds(s * ROWS_PER_S, ROWS_PER_S)])
    plsc.subcore_barrier()

    def _issue(sj, t, b):
        jg = sj * SUP + t
        pltpu.async_copy(ps_hbm.at[sidx.at[t]], ps_v.at[b], sems.at[b])
        pltpu.async_copy(pr_hbm.at[ridx.at[t]], pr_v.at[b], sems.at[b])
        pltpu.async_copy(e_hbm.at[w, jg], e_v.at[b], sems.at[b])

    def _drain(sj, t, b):
        jg = sj * SUP + t
        pltpu.make_async_copy(ps_hbm.at[sidx.at[t]], ps_v.at[b], sems.at[b]).wait()
        pltpu.make_async_copy(pr_hbm.at[ridx.at[t]], pr_v.at[b], sems.at[b]).wait()
        pltpu.make_async_copy(e_hbm.at[w, jg], e_v.at[b], sems.at[b]).wait()

    def _scatter(t, b):
        pltpu.async_copy(m_v.at[b], agg_sh.at[ridx.at[t]], sem_sc.at[b], add=True)

    def _wait_scatter(t, b):
        pltpu.make_async_copy(m_v.at[b], agg_sh.at[ridx.at[t]], sem_sc.at[b]).wait()

    def _compute(b):
        def _rows(i2, _):
            base = pl.multiple_of(i2 * 2, 2)
            for di in range(2):
                i = base + di
                for g in range(D // (2 * LANES)):
                    sl_lo = pl.ds(g * 2 * LANES, LANES)
                    sl_hi = pl.ds(g * 2 * LANES + LANES, LANES)
                    slw = pl.ds(g * LANES, LANES)
                    ew = e_v[b, i, slw]
                    pw = ps_v[b, i, slw]
                    rw = pr_v[b, i, slw]
                    lo = (
                        _xlo(pw) + _xlo(rw) + _xlo(ew)
                    )
                    hi = (
                        _xhi(pw) + _xhi(rw) + _xhi(ew)
                    )
                    m_v[b, i, sl_lo] = jnp.maximum(lo, 0.0)
                    m_v[b, i, sl_hi] = jnp.maximum(hi, 0.0)
            return 0

        lax.fori_loop(0, CHUNK // 2, _rows, 0)

    for sj in range(NSUP):
        pltpu.sync_copy(snd_hbm.at[w, sj], sidx)
        pltpu.sync_copy(rcv_hbm.at[w, sj], ridx)
        _issue(sj, 0, 0)
        _issue(sj, 1, 1)

        def _kbody(k, _, sj=sj):
            for b in (0, 1):
                t = 2 * k + b
                _drain(sj, t, b)

                @pl.when(k > 0)
                def _():
                    _wait_scatter(t - 2, b)

                _compute(b)
                _scatter(t, b)

                @pl.when(k < SPAIRS - 1)
                def _():
                    _issue(sj, t + 2, b)
            return 0

        lax.fori_loop(0, SPAIRS, _kbody, 0)
        _wait_scatter(SUP - 2, 0)
        _wait_scatter(SUP - 1, 1)

    plsc.subcore_barrier()

    pltpu.sync_copy(
        agg_sh.at[pl.ds(s * ROWS_PER_S, ROWS_PER_S)],
        out_hbm.at[c, pl.ds(s * ROWS_PER_S, ROWS_PER_S)],
    )


def _segment_messages(ps, pr, e, senders, receivers):
    mesh = plsc.VectorSubcoreMesh(
        core_axis_name="c", subcore_axis_name="s", num_cores=NC, num_subcores=NS
    )
    return pl.kernel(
        _sc_body,
        out_type=jax.ShapeDtypeStruct((NC, N_PAD, D), jnp.float32),
        mesh=mesh,
        compiler_params=pltpu.CompilerParams(use_tc_tiling_on_sc=False),
        scratch_types=[
            pltpu.VMEM_SHARED((N_PAD, D), jnp.float32),
            pltpu.VMEM((SUP, CHUNK), jnp.int32),
            pltpu.VMEM((SUP, CHUNK), jnp.int32),
            pltpu.VMEM((2, CHUNK, D // 2), jnp.int32),
            pltpu.VMEM((2, CHUNK, D // 2), jnp.int32),
            pltpu.VMEM((2, CHUNK, D // 2), jnp.int32),
            pltpu.VMEM((2, CHUNK, D), jnp.float32),
            pltpu.SemaphoreType.DMA((2,)),
            pltpu.SemaphoreType.DMA((2,)),
        ],
    )(
        ps,
        pr,
        e.reshape(NW, NCHUNKS, CHUNK, D // 2),
        senders.reshape(NW, NSUP, SUP, CHUNK),
        receivers.reshape(NW, NSUP, SUP, CHUNK),
        jnp.zeros((ROWS_PER_S, D), jnp.float32),
    )


def _pad_edges(edges, senders, receivers):
    npad = E_PAD - N_EDGES
    edges_p = jnp.concatenate([edges, jnp.zeros((npad, 16), edges.dtype)])
    senders_p = jnp.concatenate([senders, jnp.zeros((npad,), senders.dtype)])
    receivers_p = jnp.concatenate(
        [receivers, jnp.full((npad,), N_PAD - 1, receivers.dtype)]
    )
    return edges_p, senders_p, receivers_p


def kernel(nodes, edges, senders, receivers, W_msg, b_msg, W_n1, b_n1, W_n2, b_n2):
    edges, senders, receivers = _pad_edges(edges, senders, receivers)
    lo_cols = np.concatenate([np.arange(16) + 32 * g for g in range(4)])
    cols = jnp.asarray(np.concatenate([lo_cols, lo_cols + 16]))
    W_msgre = jnp.take(W_msg, cols, axis=1)
    b_msg2 = jnp.take(b_msg, cols).reshape(1, D)
    w_e_re = W_msgre[2 * D :, :]
    b1 = b_n1.reshape(1, D)
    b2 = b_n2.reshape(1, D)

    ps, pr = pl.pallas_call(
        _psr_body,
        out_shape=(
            jax.ShapeDtypeStruct((N_NODES, D // 2), jnp.int32),
            jax.ShapeDtypeStruct((N_NODES, D // 2), jnp.int32),
        ),
    )(nodes, W_msgre, b_msg2)

    eblk = 8192
    e = pl.pallas_call(
        _e_body,
        grid=(E_PAD // eblk,),
        in_specs=[
            pl.BlockSpec((eblk, 16), lambda i: (i, 0)),
            pl.BlockSpec((16, D), lambda i: (0, 0)),
        ],
        out_specs=pl.BlockSpec((eblk, D // 2), lambda i: (i, 0)),
        out_shape=jax.ShapeDtypeStruct((E_PAD, D // 2), jnp.int32),
    )(edges, w_e_re)

    agg2 = _segment_messages(ps, pr, e, senders, receivers)[:, :N_NODES, :]

    nblk = 1000
    out = pl.pallas_call(
        _final_body,
        grid=(N_NODES // nblk,),
        in_specs=[
            pl.BlockSpec((nblk, D), lambda i: (i, 0)),
            pl.BlockSpec((NC, nblk, D), lambda i: (0, i, 0)),
            pl.BlockSpec((2 * D, D), lambda i: (0, 0)),
            pl.BlockSpec((1, D), lambda i: (0, 0)),
            pl.BlockSpec((D, D), lambda i: (0, 0)),
            pl.BlockSpec((1, D), lambda i: (0, 0)),
        ],
        out_specs=pl.BlockSpec((nblk, D), lambda i: (i, 0)),
        out_shape=jax.ShapeDtypeStruct((N_NODES, D), jnp.float32),
    )(nodes, agg2, W_n1, b1, W_n2, b2)
    return out


# speedup vs baseline: 1.2971x; 1.1748x over previous
import jax
import jax.numpy as jnp
import numpy as np
from jax import lax
from jax.experimental import pallas as pl
from jax.experimental.pallas import tpu as pltpu
from jax.experimental.pallas import tpu_sc as plsc

N_NODES = 10000
N_EDGES = 320000
D = 128

NC = 2
NS = 16
NW = NC * NS
LANES = 16

N_PAD = 10240
EDGES_PER_W = N_EDGES // NW
CHUNK = 40
NCHUNKS = EDGES_PER_W // CHUNK
SUP = 50
NSUP = NCHUNKS // SUP
SPAIRS = SUP // 2
ROWS_PER_S = N_PAD // NS

def _pack_cols(x):
    lo16 = jax.lax.bitcast_convert_type(
        x[:, : D // 2].astype(jnp.bfloat16), jnp.uint16
    )
    hi16 = jax.lax.bitcast_convert_type(
        x[:, D // 2 :].astype(jnp.bfloat16), jnp.uint16
    )
    packed = (
        jax.lax.shift_left(hi16.astype(jnp.uint32), jnp.uint32(16))
        | lo16.astype(jnp.uint32)
    )
    return jax.lax.bitcast_convert_type(packed, jnp.int32)


def _psr_body(nodes_ref, w_ref, b_ref, ps_ref, pr_ref):
    n = nodes_ref[...]
    w = w_ref[...]
    ps = jnp.dot(n, w[0:D, :], preferred_element_type=jnp.float32) + b_ref[...]
    ps_ref[...] = _pack_cols(ps)
    pr = jnp.dot(n, w[D : 2 * D, :], preferred_element_type=jnp.float32)
    pr_ref[...] = _pack_cols(pr)


def _e_body(edges_ref, w_ref, e_ref):
    e = jnp.dot(edges_ref[...], w_ref[...], preferred_element_type=jnp.float32)
    e_ref[...] = _pack_cols(e)


def _final_body(nodes_ref, agg_ref, w1_ref, b1_ref, w2_ref, b2_ref, out_ref):
    n = nodes_ref[...]
    agg = agg_ref[0] + agg_ref[1]
    h = jnp.dot(n, w1_ref[0:D, :], preferred_element_type=jnp.float32)
    h = h + jnp.dot(agg, w1_ref[D:, :], preferred_element_type=jnp.float32)
    h = jnp.maximum(h + b1_ref[...], 0.0)
    h = jnp.dot(h, w2_ref[...], preferred_element_type=jnp.float32) + b2_ref[...]
    out_ref[...] = n + h


def _xlo(w):
    return jax.lax.bitcast_convert_type(jax.lax.shift_left(w, 16), jnp.float32)


def _xhi(w):
    return jax.lax.bitcast_convert_type(
        jnp.bitwise_and(w, jnp.int32(-65536)), jnp.float32
    )


def _sc_body(
    ps_hbm, pr_hbm, e_hbm, snd_hbm, rcv_hbm, z_hbm, out_hbm,
    agg_sh, sidx, ridx, ps_v, pr_v, e_v, m_v, sems, sem_sc,
):
    c = lax.axis_index("c")
    s = lax.axis_index("s")
    w = s * NC + c

    pltpu.sync_copy(z_hbm, agg_sh.at[pl.ds(s * ROWS_PER_S, ROWS_PER_S)])
    plsc.subcore_barrier()

    def _ebase(sj, t):
        return pl.multiple_of(
            w * EDGES_PER_W + (sj * SUP + t) * CHUNK, 8
        )

    def _issue(sj, t, b):
        pltpu.async_copy(ps_hbm.at[sidx.at[t]], ps_v.at[b], sems.at[b])
        pltpu.async_copy(pr_hbm.at[ridx.at[t]], pr_v.at[b], sems.at[b])
        pltpu.async_copy(e_hbm.at[pl.ds(_ebase(sj, t), CHUNK)], e_v.at[b], sems.at[b])

    def _drain(sj, t, b):
        pltpu.make_async_copy(ps_hbm.at[sidx.at[t]], ps_v.at[b], sems.at[b]).wait()
        pltpu.make_async_copy(pr_hbm.at[ridx.at[t]], pr_v.at[b], sems.at[b]).wait()
        pltpu.make_async_copy(
            e_hbm.at[pl.ds(_ebase(sj, t), CHUNK)], e_v.at[b], sems.at[b]
        ).wait()

    def _scatter(t, b):
        pltpu.async_copy(m_v.at[b], agg_sh.at[ridx.at[t]], sem_sc.at[b], add=True)

    def _wait_scatter(t, b):
        pltpu.make_async_copy(m_v.at[b], agg_sh.at[ridx.at[t]], sem_sc.at[b]).wait()

    def _compute(b):
        def _rows(i2, _):
            base = pl.multiple_of(i2 * 2, 2)
            for di in range(2):
                i = base + di
                for g in range(D // (2 * LANES)):
                    sl_lo = pl.ds(g * 2 * LANES, LANES)
                    sl_hi = pl.ds(g * 2 * LANES + LANES, LANES)
                    slw = pl.ds(g * LANES, LANES)
                    ew = e_v[b, i, slw]
                    pw = ps_v[b, i, slw]
                    rw = pr_v[b, i, slw]
                    lo = (
                        _xlo(pw) + _xlo(rw) + _xlo(ew)
                    )
                    hi = (
                        _xhi(pw) + _xhi(rw) + _xhi(ew)
                    )
                    m_v[b, i, sl_lo] = jnp.maximum(lo, 0.0)
                    m_v[b, i, sl_hi] = jnp.maximum(hi, 0.0)
            return 0

        lax.fori_loop(0, CHUNK // 2, _rows, 0)

    for sj in range(NSUP):
        pltpu.sync_copy(snd_hbm.at[w, sj], sidx)
        pltpu.sync_copy(rcv_hbm.at[w, sj], ridx)
        _issue(sj, 0, 0)
        _issue(sj, 1, 1)

        def _kbody(k, _, sj=sj):
            for b in (0, 1):
                t = 2 * k + b
                _drain(sj, t, b)

                @pl.when(k > 0)
                def _():
                    _wait_scatter(t - 2, b)

                _compute(b)
                _scatter(t, b)

                @pl.when(k < SPAIRS - 1)
                def _():
                    _issue(sj, t + 2, b)
            return 0

        lax.fori_loop(0, SPAIRS, _kbody, 0)
        _wait_scatter(SUP - 2, 0)
        _wait_scatter(SUP - 1, 1)

    plsc.subcore_barrier()

    pltpu.sync_copy(
        agg_sh.at[pl.ds(s * ROWS_PER_S, ROWS_PER_S)],
        out_hbm.at[c, pl.ds(s * ROWS_PER_S, ROWS_PER_S)],
    )


def _segment_messages(ps, pr, e, senders, receivers):
    mesh = plsc.VectorSubcoreMesh(
        core_axis_name="c", subcore_axis_name="s", num_cores=NC, num_subcores=NS
    )
    return pl.kernel(
        _sc_body,
        out_type=jax.ShapeDtypeStruct((NC, N_PAD, D), jnp.float32),
        mesh=mesh,
        compiler_params=pltpu.CompilerParams(use_tc_tiling_on_sc=False),
        scratch_types=[
            pltpu.VMEM_SHARED((N_PAD, D), jnp.float32),
            pltpu.VMEM((SUP, CHUNK), jnp.int32),
            pltpu.VMEM((SUP, CHUNK), jnp.int32),
            pltpu.VMEM((2, CHUNK, D // 2), jnp.int32),
            pltpu.VMEM((2, CHUNK, D // 2), jnp.int32),
            pltpu.VMEM((2, CHUNK, D // 2), jnp.int32),
            pltpu.VMEM((2, CHUNK, D), jnp.float32),
            pltpu.SemaphoreType.DMA((2,)),
            pltpu.SemaphoreType.DMA((2,)),
        ],
    )(
        ps,
        pr,
        e,
        senders.reshape(NW, NSUP, SUP, CHUNK),
        receivers.reshape(NW, NSUP, SUP, CHUNK),
        jnp.zeros((ROWS_PER_S, D), jnp.float32),
    )


def kernel(nodes, edges, senders, receivers, W_msg, b_msg, W_n1, b_n1, W_n2, b_n2):
    lo_cols = np.concatenate([np.arange(16) + 32 * g for g in range(4)])
    cols = jnp.asarray(np.concatenate([lo_cols, lo_cols + 16]))
    W_msgre = jnp.take(W_msg, cols, axis=1)
    b_msg2 = jnp.take(b_msg, cols).reshape(1, D)
    w_e_re = W_msgre[2 * D :, :]
    b1 = b_n1.reshape(1, D)
    b2 = b_n2.reshape(1, D)

    ps, pr = pl.pallas_call(
        _psr_body,
        out_shape=(
            jax.ShapeDtypeStruct((N_NODES, D // 2), jnp.int32),
            jax.ShapeDtypeStruct((N_NODES, D // 2), jnp.int32),
        ),
    )(nodes, W_msgre, b_msg2)

    eblk = 8000
    e = pl.pallas_call(
        _e_body,
        grid=(N_EDGES // eblk,),
        in_specs=[
            pl.BlockSpec((eblk, 16), lambda i: (i, 0)),
            pl.BlockSpec((16, D), lambda i: (0, 0)),
        ],
        out_specs=pl.BlockSpec((eblk, D // 2), lambda i: (i, 0)),
        out_shape=jax.ShapeDtypeStruct((N_EDGES, D // 2), jnp.int32),
    )(edges, w_e_re)

    agg2 = _segment_messages(ps, pr, e, senders, receivers)[:, :N_NODES, :]

    nblk = 1000
    out = pl.pallas_call(
        _final_body,
        grid=(N_NODES // nblk,),
        in_specs=[
            pl.BlockSpec((nblk, D), lambda i: (i, 0)),
            pl.BlockSpec((NC, nblk, D), lambda i: (0, i, 0)),
            pl.BlockSpec((2 * D, D), lambda i: (0, 0)),
            pl.BlockSpec((1, D), lambda i: (0, 0)),
            pl.BlockSpec((D, D), lambda i: (0, 0)),
            pl.BlockSpec((1, D), lambda i: (0, 0)),
        ],
        out_specs=pl.BlockSpec((nblk, D), lambda i: (i, 0)),
        out_shape=jax.ShapeDtypeStruct((N_NODES, D), jnp.float32),
    )(nodes, agg2, W_n1, b1, W_n2, b2)
    return out
